# baseline (speedup 1.0000x reference)
import jax
import jax.numpy as jnp
from jax import lax
from jax.experimental import pallas as pl
from jax.experimental.pallas import tpu as pltpu

DIM = 64
NUM_ITEM = 1000
NI_PAD = 1024
BATCH = 1024

_BB = 128
_IT = 128
_NI = BATCH // _BB
_NJ = NI_PAD // _IT


def _body(wcol_sref, clane_ref, items_ref, atf_ref, at_ref, buy_ref,
          table_ref, rank_ref, pred_ref, wbuf, wsem, u_scr):
    i = pl.program_id(0)
    j = pl.program_id(1)
    n = pl.num_programs(0)

    def fire(tile, slot):
        for b in range(_BB):
            col0 = pl.multiple_of(wcol_sref[tile * _BB + b], 128)
            pltpu.make_async_copy(
                table_ref.at[:, pl.ds(col0, 128)], wbuf.at[slot, b],
                wsem.at[slot],
            ).start()

    @pl.when(j == 0)
    def _():
        @pl.when(i == 0)
        def _():
            fire(0, 0)

        @pl.when(i + 1 < n)
        def _():
            fire(i + 1, (i + 1) % 2)

        sl = i % 2
        for b in range(_BB):
            pltpu.make_async_copy(
                table_ref.at[:, pl.ds(0, 128)], wbuf.at[sl, b], wsem.at[sl]
            ).wait()
        w = wbuf[sl]
        c_b = clane_ref[...]
        lane = lax.broadcasted_iota(jnp.int32, (_BB, DIM, 128), 2)
        mask = lane == c_b[:, None, None]
        u_scr[...] = jnp.where(mask, w, 0.0).sum(axis=2)

    u = u_scr[...]
    at = at_ref[...]
    buy = buy_ref[0:1, :]
    up = u + buy

    acc = jnp.zeros((_BB, _IT), jnp.float32)
    for d in range(DIM):
        col = up[:, d:d + 1]
        row = at[d:d + 1, :]
        acc = acc + jnp.abs(col - row)

    scores = jnp.dot(u, at, preferred_element_type=jnp.float32)
    rank_ref[...] = jax.nn.sigmoid(acc + scores)

    @pl.when(j == 0)
    def _():
        atf = atf_ref[...]
        items = items_ref[...]
        cols = lax.broadcasted_iota(jnp.int32, (_BB, NI_PAD), 1)
        onehot = (cols == items[:, None]).astype(jnp.float32)
        ie = lax.dot_general(
            onehot, atf, (((1,), (1,)), ((), ())),
            preferred_element_type=jnp.float32,
        )
        ps = jnp.sum(u * ie, axis=1)
        pd = jnp.sum(jnp.abs(up - ie), axis=1)
        pred_ref[...] = jax.nn.sigmoid(pd + ps)


def _fused(table_t, wcol, clane, items, a_t, buy8):
    rank, pred = pl.pallas_call(
        _body,
        grid_spec=pltpu.PrefetchScalarGridSpec(
            num_scalar_prefetch=1,
            grid=(_NI, _NJ),
            in_specs=[
                pl.BlockSpec((_BB,), lambda i, j, s: (i,)),
                pl.BlockSpec((_BB,), lambda i, j, s: (i,)),
                pl.BlockSpec((DIM, NI_PAD), lambda i, j, s: (0, 0)),
                pl.BlockSpec((DIM, _IT), lambda i, j, s: (0, j)),
                pl.BlockSpec((8, DIM), lambda i, j, s: (0, 0)),
                pl.BlockSpec(memory_space=pltpu.HBM),
            ],
            out_specs=[
                pl.BlockSpec((_BB, _IT), lambda i, j, s: (i, j)),
                pl.BlockSpec((_BB,), lambda i, j, s: (i,)),
            ],
            scratch_shapes=[
                pltpu.VMEM((2, _BB, DIM, 128), jnp.float32),
                pltpu.SemaphoreType.DMA((2,)),
                pltpu.VMEM((_BB, DIM), jnp.float32),
            ],
        ),
        out_shape=[
            jax.ShapeDtypeStruct((BATCH, NUM_ITEM), jnp.float32),
            jax.ShapeDtypeStruct((BATCH,), jnp.float32),
        ],
    )(wcol, clane, items, a_t, a_t, buy8, table_t)
    return rank, pred


def kernel(users, items, entity_table, relation_table):
    users = users.astype(jnp.int32)
    items = items.astype(jnp.int32)
    table_t = entity_table.T
    wcol = (users >> 7) << 7
    clane = users & 127
    a_t = table_t[:, :NI_PAD]
    buy8 = jnp.broadcast_to(relation_table[-1], (8, DIM))

    rank, pred = _fused(table_t, wcol, clane, items, a_t, buy8)
    return (pred, rank)


# speedup vs baseline: 1.8841x; 1.8841x over previous
import jax
import jax.numpy as jnp
from jax import lax
from jax.experimental import pallas as pl
from jax.experimental.pallas import tpu as pltpu

DIM = 64
NUM_ITEM = 1000
NI_PAD = 1024
BATCH = 1024

_BB = 128
_IT = 512
_NI = BATCH // _BB
_NJ = NI_PAD // _IT


def _body(wcol_sref, clane_ref, items_ref, atf_ref, at_ref, buy_ref,
          table_ref, rank_ref, pred_ref, wbuf, wsem, u_scr):
    i = pl.program_id(0)
    j = pl.program_id(1)
    n = pl.num_programs(0)

    def fire(tile, slot):
        for b in range(_BB):
            col0 = pl.multiple_of(wcol_sref[tile * _BB + b], 128)
            pltpu.make_async_copy(
                table_ref.at[:, pl.ds(col0, 128)], wbuf.at[slot, b],
                wsem.at[slot],
            ).start()

    @pl.when(j == 0)
    def _():
        @pl.when(i == 0)
        def _():
            fire(0, 0)

        @pl.when(i + 1 < n)
        def _():
            fire(i + 1, (i + 1) % 2)

        sl = i % 2
        for b in range(_BB):
            pltpu.make_async_copy(
                table_ref.at[:, pl.ds(0, 128)], wbuf.at[sl, b], wsem.at[sl]
            ).wait()
        w = wbuf[sl]
        c_b = clane_ref[...]
        lane = lax.broadcasted_iota(jnp.int32, (_BB, DIM, 128), 2)
        mask = lane == c_b[:, None, None]
        u_scr[...] = jnp.where(mask, w, 0.0).sum(axis=2)

    u = u_scr[...]
    at = at_ref[...]
    buy = buy_ref[0:1, :]
    up = u + buy

    acc = jnp.zeros((_BB, _IT), jnp.float32)
    for d in range(DIM):
        col = up[:, d:d + 1]
        row = at[d:d + 1, :]
        acc = acc + jnp.abs(col - row)

    scores = jnp.dot(u, at, preferred_element_type=jnp.float32)
    rank_ref[...] = jax.nn.sigmoid(acc + scores)

    @pl.when(j == 0)
    def _():
        atf = atf_ref[...]
        items = items_ref[...]
        cols = lax.broadcasted_iota(jnp.int32, (_BB, NI_PAD), 1)
        onehot = (cols == items[:, None]).astype(jnp.float32)
        ie = lax.dot_general(
            onehot, atf, (((1,), (1,)), ((), ())),
            preferred_element_type=jnp.float32,
        )
        ps = jnp.sum(u * ie, axis=1)
        pd = jnp.sum(jnp.abs(up - ie), axis=1)
        pred_ref[...] = jax.nn.sigmoid(pd + ps)


def _fused(table_t, wcol, clane, items, a_t, buy8):
    rank, pred = pl.pallas_call(
        _body,
        grid_spec=pltpu.PrefetchScalarGridSpec(
            num_scalar_prefetch=1,
            grid=(_NI, _NJ),
            in_specs=[
                pl.BlockSpec((_BB,), lambda i, j, s: (i,)),
                pl.BlockSpec((_BB,), lambda i, j, s: (i,)),
                pl.BlockSpec((DIM, NI_PAD), lambda i, j, s: (0, 0)),
                pl.BlockSpec((DIM, _IT), lambda i, j, s: (0, j)),
                pl.BlockSpec((8, DIM), lambda i, j, s: (0, 0)),
                pl.BlockSpec(memory_space=pltpu.HBM),
            ],
            out_specs=[
                pl.BlockSpec((_BB, _IT), lambda i, j, s: (i, j)),
                pl.BlockSpec((_BB,), lambda i, j, s: (i,)),
            ],
            scratch_shapes=[
                pltpu.VMEM((2, _BB, DIM, 128), jnp.float32),
                pltpu.SemaphoreType.DMA((2,)),
                pltpu.VMEM((_BB, DIM), jnp.float32),
            ],
        ),
        out_shape=[
            jax.ShapeDtypeStruct((BATCH, NUM_ITEM), jnp.float32),
            jax.ShapeDtypeStruct((BATCH,), jnp.float32),
        ],
    )(wcol, clane, items, a_t, a_t, buy8, table_t)
    return rank, pred


def kernel(users, items, entity_table, relation_table):
    users = users.astype(jnp.int32)
    items = items.astype(jnp.int32)
    table_t = entity_table.T
    wcol = (users >> 7) << 7
    clane = users & 127
    a_t = table_t[:, :NI_PAD]
    buy8 = jnp.broadcast_to(relation_table[-1], (8, DIM))

    rank, pred = _fused(table_t, wcol, clane, items, a_t, buy8)
    return (pred, rank)


# speedup vs baseline: 1.9238x; 1.0210x over previous
import jax
import jax.numpy as jnp
from jax import lax
from jax.experimental import pallas as pl
from jax.experimental.pallas import tpu as pltpu

DIM = 64
NUM_ITEM = 1000
NI_PAD = 1024
BATCH = 1024

_BB = 128
_IT = 1024
_NI = BATCH // _BB
_NJ = NI_PAD // _IT


def _body(wcol_sref, clane_ref, items_ref, atf_ref, at_ref, buy_ref,
          table_ref, rank_ref, pred_ref, wbuf, wsem, u_scr):
    i = pl.program_id(0)
    j = pl.program_id(1)
    n = pl.num_programs(0)

    def fire(tile, slot):
        for b in range(_BB):
            col0 = pl.multiple_of(wcol_sref[tile * _BB + b], 128)
            pltpu.make_async_copy(
                table_ref.at[:, pl.ds(col0, 128)], wbuf.at[slot, b],
                wsem.at[slot],
            ).start()

    @pl.when(j == 0)
    def _():
        @pl.when(i == 0)
        def _():
            fire(0, 0)

        @pl.when(i + 1 < n)
        def _():
            fire(i + 1, (i + 1) % 2)

        sl = i % 2
        for b in range(_BB):
            pltpu.make_async_copy(
                table_ref.at[:, pl.ds(0, 128)], wbuf.at[sl, b], wsem.at[sl]
            ).wait()
        w = wbuf[sl]
        c_b = clane_ref[...]
        lane = lax.broadcasted_iota(jnp.int32, (_BB, DIM, 128), 2)
        mask = lane == c_b[:, None, None]
        u_scr[...] = jnp.where(mask, w, 0.0).sum(axis=2)

    u = u_scr[...]
    at = at_ref[...]
    buy = buy_ref[0:1, :]
    up = u + buy

    acc = jnp.zeros((_BB, _IT), jnp.float32)
    for d in range(DIM):
        col = up[:, d:d + 1]
        row = at[d:d + 1, :]
        acc = acc + jnp.abs(col - row)

    scores = jnp.dot(u, at, preferred_element_type=jnp.float32)
    rank_ref[...] = jax.nn.sigmoid(acc + scores)

    @pl.when(j == 0)
    def _():
        atf = atf_ref[...]
        items = items_ref[...]
        cols = lax.broadcasted_iota(jnp.int32, (_BB, NI_PAD), 1)
        onehot = (cols == items[:, None]).astype(jnp.float32)
        ie = lax.dot_general(
            onehot, atf, (((1,), (1,)), ((), ())),
            preferred_element_type=jnp.float32,
        )
        ps = jnp.sum(u * ie, axis=1)
        pd = jnp.sum(jnp.abs(up - ie), axis=1)
        pred_ref[...] = jax.nn.sigmoid(pd + ps)


def _fused(table_t, wcol, clane, items, a_t, buy8):
    rank, pred = pl.pallas_call(
        _body,
        grid_spec=pltpu.PrefetchScalarGridSpec(
            num_scalar_prefetch=1,
            grid=(_NI, _NJ),
            in_specs=[
                pl.BlockSpec((_BB,), lambda i, j, s: (i,)),
                pl.BlockSpec((_BB,), lambda i, j, s: (i,)),
                pl.BlockSpec((DIM, NI_PAD), lambda i, j, s: (0, 0)),
                pl.BlockSpec((DIM, _IT), lambda i, j, s: (0, j)),
                pl.BlockSpec((8, DIM), lambda i, j, s: (0, 0)),
                pl.BlockSpec(memory_space=pltpu.HBM),
            ],
            out_specs=[
                pl.BlockSpec((_BB, _IT), lambda i, j, s: (i, j)),
                pl.BlockSpec((_BB,), lambda i, j, s: (i,)),
            ],
            scratch_shapes=[
                pltpu.VMEM((2, _BB, DIM, 128), jnp.float32),
                pltpu.SemaphoreType.DMA((2,)),
                pltpu.VMEM((_BB, DIM), jnp.float32),
            ],
        ),
        out_shape=[
            jax.ShapeDtypeStruct((BATCH, NUM_ITEM), jnp.float32),
            jax.ShapeDtypeStruct((BATCH,), jnp.float32),
        ],
    )(wcol, clane, items, a_t, a_t, buy8, table_t)
    return rank, pred


def kernel(users, items, entity_table, relation_table):
    users = users.astype(jnp.int32)
    items = items.astype(jnp.int32)
    table_t = entity_table.T
    wcol = (users >> 7) << 7
    clane = users & 127
    a_t = table_t[:, :NI_PAD]
    buy8 = jnp.broadcast_to(relation_table[-1], (8, DIM))

    rank, pred = _fused(table_t, wcol, clane, items, a_t, buy8)
    return (pred, rank)


# speedup vs baseline: 1.9238x; 1.0000x over previous
import jax
import jax.numpy as jnp
from jax import lax
from jax.experimental import pallas as pl
from jax.experimental.pallas import tpu as pltpu

DIM = 64
NUM_ITEM = 1000
NI_PAD = 1024
BATCH = 1024

_BB = 256
_IT = 1024
_NI = BATCH // _BB
_NJ = NI_PAD // _IT


def _body(wcol_sref, clane_ref, items_ref, atf_ref, at_ref, buy_ref,
          table_ref, rank_ref, pred_ref, wbuf, wsem, u_scr):
    i = pl.program_id(0)
    j = pl.program_id(1)
    n = pl.num_programs(0)

    def fire(tile, slot):
        for b in range(_BB):
            col0 = pl.multiple_of(wcol_sref[tile * _BB + b], 128)
            pltpu.make_async_copy(
                table_ref.at[:, pl.ds(col0, 128)], wbuf.at[slot, b],
                wsem.at[slot],
            ).start()

    @pl.when(j == 0)
    def _():
        @pl.when(i == 0)
        def _():
            fire(0, 0)

        @pl.when(i + 1 < n)
        def _():
            fire(i + 1, (i + 1) % 2)

        sl = i % 2
        for b in range(_BB):
            pltpu.make_async_copy(
                table_ref.at[:, pl.ds(0, 128)], wbuf.at[sl, b], wsem.at[sl]
            ).wait()
        w = wbuf[sl]
        c_b = clane_ref[...]
        lane = lax.broadcasted_iota(jnp.int32, (_BB, DIM, 128), 2)
        mask = lane == c_b[:, None, None]
        u_scr[...] = jnp.where(mask, w, 0.0).sum(axis=2)

    u = u_scr[...]
    at = at_ref[...]
    buy = buy_ref[0:1, :]
    up = u + buy

    acc = jnp.zeros((_BB, _IT), jnp.float32)
    for d in range(DIM):
        col = up[:, d:d + 1]
        row = at[d:d + 1, :]
        acc = acc + jnp.abs(col - row)

    scores = jnp.dot(u, at, preferred_element_type=jnp.float32)
    rank_ref[...] = jax.nn.sigmoid(acc + scores)

    @pl.when(j == 0)
    def _():
        atf = atf_ref[...]
        items = items_ref[...]
        cols = lax.broadcasted_iota(jnp.int32, (_BB, NI_PAD), 1)
        onehot = (cols == items[:, None]).astype(jnp.float32)
        ie = lax.dot_general(
            onehot, atf, (((1,), (1,)), ((), ())),
            preferred_element_type=jnp.float32,
        )
        ps = jnp.sum(u * ie, axis=1)
        pd = jnp.sum(jnp.abs(up - ie), axis=1)
        pred_ref[...] = jax.nn.sigmoid(pd + ps)


def _fused(table_t, wcol, clane, items, a_t, buy8):
    rank, pred = pl.pallas_call(
        _body,
        grid_spec=pltpu.PrefetchScalarGridSpec(
            num_scalar_prefetch=1,
            grid=(_NI, _NJ),
            in_specs=[
                pl.BlockSpec((_BB,), lambda i, j, s: (i,)),
                pl.BlockSpec((_BB,), lambda i, j, s: (i,)),
                pl.BlockSpec((DIM, NI_PAD), lambda i, j, s: (0, 0)),
                pl.BlockSpec((DIM, _IT), lambda i, j, s: (0, j)),
                pl.BlockSpec((8, DIM), lambda i, j, s: (0, 0)),
                pl.BlockSpec(memory_space=pltpu.HBM),
            ],
            out_specs=[
                pl.BlockSpec((_BB, _IT), lambda i, j, s: (i, j)),
                pl.BlockSpec((_BB,), lambda i, j, s: (i,)),
            ],
            scratch_shapes=[
                pltpu.VMEM((2, _BB, DIM, 128), jnp.float32),
                pltpu.SemaphoreType.DMA((2,)),
                pltpu.VMEM((_BB, DIM), jnp.float32),
            ],
        ),
        out_shape=[
            jax.ShapeDtypeStruct((BATCH, NUM_ITEM), jnp.float32),
            jax.ShapeDtypeStruct((BATCH,), jnp.float32),
        ],
    )(wcol, clane, items, a_t, a_t, buy8, table_t)
    return rank, pred


def kernel(users, items, entity_table, relation_table):
    users = users.astype(jnp.int32)
    items = items.astype(jnp.int32)
    table_t = entity_table.T
    wcol = (users >> 7) << 7
    clane = users & 127
    a_t = table_t[:, :NI_PAD]
    buy8 = jnp.broadcast_to(relation_table[-1], (8, DIM))

    rank, pred = _fused(table_t, wcol, clane, items, a_t, buy8)
    return (pred, rank)


# speedup vs baseline: 2.1273x; 1.1058x over previous
import jax
import jax.numpy as jnp
from jax import lax
from jax.experimental import pallas as pl
from jax.experimental.pallas import tpu as pltpu

DIM = 64
NUM_ITEM = 1000
NI_PAD = 1024
BATCH = 1024

_BB = 128
_NI = BATCH // _BB


def _body(users_sref, uvec_ref, items_ref, at_ref, rel_ref,
          table_ref, rank_ref, pred_ref, wbuf, wsem, u_scr):
    i = pl.program_id(0)
    n = pl.num_programs(0)

    def fire(tile, slot):
        for b in range(_BB):
            r = users_sref[tile * _BB + b]
            col0 = pl.multiple_of((r >> 7) << 7, 128)
            pltpu.make_async_copy(
                table_ref.at[:, pl.ds(col0, 128)], wbuf.at[slot, b],
                wsem.at[slot],
            ).start()

    @pl.when(i == 0)
    def _():
        fire(0, 0)

    sl = i % 2
    for b in range(_BB):
        pltpu.make_async_copy(
            table_ref.at[:, pl.ds(0, 128)], wbuf.at[sl, b], wsem.at[sl]
        ).wait()
    w = wbuf[sl]
    c_b = uvec_ref[...] & 127
    lane = lax.broadcasted_iota(jnp.int32, (_BB, DIM, 128), 2)
    mask = lane == c_b[:, None, None]
    u_scr[...] = jnp.where(mask, w, 0.0).sum(axis=2)

    @pl.when(i + 1 < n)
    def _():
        fire(i + 1, (i + 1) % 2)

    u = u_scr[...]
    at = at_ref[...]
    buy = rel_ref[7:8, :]
    up = u + buy

    acc = jnp.zeros((_BB, NI_PAD), jnp.float32)
    for d in range(DIM):
        col = up[:, d:d + 1]
        row = at[d:d + 1, :]
        acc = acc + jnp.abs(col - row)

    scores = jnp.dot(u, at, preferred_element_type=jnp.float32)
    rank_ref[...] = jax.nn.sigmoid(acc + scores)

    items = items_ref[...]
    cols = lax.broadcasted_iota(jnp.int32, (_BB, NI_PAD), 1)
    onehot = (cols == items[:, None]).astype(jnp.float32)
    ie = lax.dot_general(
        onehot, at, (((1,), (1,)), ((), ())),
        preferred_element_type=jnp.float32,
    )
    ps = jnp.sum(u * ie, axis=1)
    pd = jnp.sum(jnp.abs(up - ie), axis=1)
    pred_ref[...] = jax.nn.sigmoid(pd + ps)


def _fused(table_t, users, items, relation_table):
    rank, pred = pl.pallas_call(
        _body,
        grid_spec=pltpu.PrefetchScalarGridSpec(
            num_scalar_prefetch=1,
            grid=(_NI,),
            in_specs=[
                pl.BlockSpec((_BB,), lambda i, s: (i,)),
                pl.BlockSpec((_BB,), lambda i, s: (i,)),
                pl.BlockSpec((DIM, NI_PAD), lambda i, s: (0, 0)),
                pl.BlockSpec((8, DIM), lambda i, s: (7, 0)),
                pl.BlockSpec(memory_space=pltpu.HBM),
            ],
            out_specs=[
                pl.BlockSpec((_BB, NI_PAD), lambda i, s: (i, 0)),
                pl.BlockSpec((_BB,), lambda i, s: (i,)),
            ],
            scratch_shapes=[
                pltpu.VMEM((2, _BB, DIM, 128), jnp.float32),
                pltpu.SemaphoreType.DMA((2,)),
                pltpu.VMEM((_BB, DIM), jnp.float32),
            ],
        ),
        out_shape=[
            jax.ShapeDtypeStruct((BATCH, NUM_ITEM), jnp.float32),
            jax.ShapeDtypeStruct((BATCH,), jnp.float32),
        ],
    )(users, users, items, table_t, relation_table, table_t)
    return rank, pred


def kernel(users, items, entity_table, relation_table):
    users = users.astype(jnp.int32)
    items = items.astype(jnp.int32)
    table_t = entity_table.T
    rank, pred = _fused(table_t, users, items, relation_table)
    return (pred, rank)
